# baseline (speedup 1.0000x reference)
import jax
import jax.numpy as jnp
from jax.experimental import pallas as pl
from jax.experimental.pallas import tpu as pltpu


def _round_up(x, m):
    return ((x + m - 1) // m) * m


def _pick_tile_n(n, prefer=2048):
    if n <= prefer:
        return n
    for cand in (2048, 1024, 512, 384, 256, 128):
        if cand <= prefer and n % cand == 0:
            return cand
    return n


def _fused_kernel(idx_ref, table_ref, w1_ref, b1_ref, w2_ref, b2_ref, o_ref):
    h = jnp.dot(table_ref[...].astype(jnp.bfloat16),
                w1_ref[...].astype(jnp.bfloat16),
                preferred_element_type=jnp.float32)
    ht = jnp.tanh(h + b1_ref[...]).astype(jnp.bfloat16)
    c = jnp.dot(ht, w2_ref[...].astype(jnp.bfloat16),
                preferred_element_type=jnp.float32) + b2_ref[...]
    ids = idx_ref[...]
    iota = jax.lax.broadcasted_iota(
        jnp.int32, (ids.shape[0], table_ref.shape[0]), 1)
    onehot = (ids == iota).astype(jnp.bfloat16)
    o_ref[...] = jnp.dot(onehot, c.astype(jnp.bfloat16),
                         preferred_element_type=jnp.float32)


def kernel(prefix, embedding, w1, b1, w2, b2):
    B, L = prefix.shape
    M = B * L
    P, H = embedding.shape
    PH = w1.shape[1]
    N = w2.shape[1]

    M_pad = _round_up(M, 8)
    idx2d = jnp.pad(prefix.reshape(-1).astype(jnp.int32),
                    (0, M_pad - M)).reshape(M_pad, 1)

    TN = 4096 if N % 4096 == 0 else _pick_tile_n(N)
    TM = 1024 if (TN == 4096 and M_pad % 1024 == 0) else M_pad
    grid = (N // TN, M_pad // TM)
    out = pl.pallas_call(
        _fused_kernel,
        out_shape=jax.ShapeDtypeStruct((M_pad, N), jnp.float32),
        grid_spec=pltpu.PrefetchScalarGridSpec(
            num_scalar_prefetch=0,
            grid=grid,
            in_specs=[
                pl.BlockSpec((TM, 1), lambda j, i: (i, 0)),
                pl.BlockSpec((P, H), lambda j, i: (0, 0)),
                pl.BlockSpec((H, PH), lambda j, i: (0, 0)),
                pl.BlockSpec((1, PH), lambda j, i: (0, 0)),
                pl.BlockSpec((PH, TN), lambda j, i: (0, j)),
                pl.BlockSpec((1, TN), lambda j, i: (0, j)),
            ],
            out_specs=pl.BlockSpec((TM, TN), lambda j, i: (i, j)),
        ),
        compiler_params=pltpu.CompilerParams(
            dimension_semantics=("parallel", "arbitrary")),
    )(idx2d, embedding, w1, b1, w2, b2)
    return out[:M].reshape(B, L, N)


# speedup vs baseline: 1.0285x; 1.0285x over previous
import jax
import jax.numpy as jnp
from jax.experimental import pallas as pl
from jax.experimental.pallas import tpu as pltpu


def _round_up(x, m):
    return ((x + m - 1) // m) * m


def _pick_tile_n(n, prefer=2048):
    if n <= prefer:
        return n
    for cand in (2048, 1024, 512, 384, 256, 128):
        if cand <= prefer and n % cand == 0:
            return cand
    return n


def _fused_kernel(idx_ref, table_ref, w1_ref, b1_ref, w2_ref, b2_ref, o_ref):
    h = jnp.dot(table_ref[...].astype(jnp.bfloat16),
                w1_ref[...].astype(jnp.bfloat16),
                preferred_element_type=jnp.float32)
    ht = jnp.tanh(h + b1_ref[...]).astype(jnp.bfloat16)
    c = jnp.dot(ht, w2_ref[...].astype(jnp.bfloat16),
                preferred_element_type=jnp.float32) + b2_ref[...]
    ids = idx_ref[...]
    iota = jax.lax.broadcasted_iota(
        jnp.int32, (ids.shape[0], table_ref.shape[0]), 1)
    onehot = (ids == iota).astype(jnp.bfloat16)
    o_ref[...] = jnp.dot(onehot, c.astype(jnp.bfloat16),
                         preferred_element_type=jnp.float32)


def kernel(prefix, embedding, w1, b1, w2, b2):
    B, L = prefix.shape
    M = B * L
    P, H = embedding.shape
    PH = w1.shape[1]
    N = w2.shape[1]

    M_pad = _round_up(M, 8)
    idx2d = jnp.pad(prefix.reshape(-1).astype(jnp.int32),
                    (0, M_pad - M)).reshape(M_pad, 1)

    TN = _pick_tile_n(N)
    grid = (N // TN,)
    out = pl.pallas_call(
        _fused_kernel,
        out_shape=jax.ShapeDtypeStruct((M_pad, N), jnp.float32),
        grid_spec=pltpu.PrefetchScalarGridSpec(
            num_scalar_prefetch=0,
            grid=grid,
            in_specs=[
                pl.BlockSpec((M_pad, 1), lambda j: (0, 0)),
                pl.BlockSpec((P, H), lambda j: (0, 0)),
                pl.BlockSpec((H, PH), lambda j: (0, 0)),
                pl.BlockSpec((1, PH), lambda j: (0, 0)),
                pl.BlockSpec((PH, TN), lambda j: (0, j)),
                pl.BlockSpec((1, TN), lambda j: (0, j)),
            ],
            out_specs=pl.BlockSpec((M_pad, TN), lambda j: (0, j)),
        ),
        compiler_params=pltpu.CompilerParams(
            dimension_semantics=("parallel",)),
    )(idx2d, embedding, w1, b1, w2, b2)
    return out[:M].reshape(B, L, N)
